# Initial kernel scaffold; baseline (speedup 1.0000x reference)
import jax
import jax.numpy as jnp
from jax.experimental import pallas as pl


def kernel(feat_user, feat_item, edge_index, pos_src, pos_dst, neg_src, neg_dst, alpha_T, W_user, b_user, W_item, b_item, W_gat, attn_l, attn_r, b_gat, W_pred, b_pred):
    raise NotImplementedError("write your pallas kernel here")


# speedup vs baseline: 11.4241x; 11.4241x over previous
import functools

import jax
import jax.numpy as jnp
from jax import lax
from jax.experimental import pallas as pl
from jax.experimental.pallas import tpu as pltpu
from jax.experimental.pallas import tpu_sc as plsc

N_USER = 5000
N_ITEM = 5000
N_NODES = N_USER + N_ITEM
E_RAW = 320000
E_SL = E_RAW + N_NODES
P = 8192

NC = 2
NS = 16
NW = NC * NS
L = 16

E_PAD = 344064
EPW = E_PAD // NW
G = 64
NCH = EPW // G
NSEG = 3
CPS = NCH // NSEG
NDV = N_NODES // L
ND_PAD = 10240
NDR = ND_PAD // 128
ROWS_PT = N_NODES // NW
ROWS_PS = ND_PAD // NS
ZB = 128
B_SCORE = 4 * P
BPW = B_SCORE // NW
GC = 128
NCHC = BPW // GC

_mesh = plsc.VectorSubcoreMesh(core_axis_name="c", subcore_axis_name="s")


def _fori(lo, hi, body):
    lax.fori_loop(jnp.int32(lo), jnp.int32(hi), body, 0)


def _tc_pre_body(fu_ref, fi_ref, wu_ref, bu_ref, wi_ref, bi_ref, wg_ref,
                 aa_ref, h0_ref, elr_ref, c_ref):
    wg = wg_ref[...]
    m_u = jnp.dot(wu_ref[...], wg, preferred_element_type=jnp.float32)
    m_i = jnp.dot(wi_ref[...], wg, preferred_element_type=jnp.float32)
    cb_u = jnp.dot(bu_ref[...], wg, preferred_element_type=jnp.float32)
    cb_i = jnp.dot(bi_ref[...], wg, preferred_element_type=jnp.float32)
    h0u = jnp.dot(fu_ref[...], m_u, preferred_element_type=jnp.float32) + cb_u
    h0i = jnp.dot(fi_ref[...], m_i, preferred_element_type=jnp.float32) + cb_i
    h0_ref[0:N_USER, :] = h0u
    h0_ref[N_USER:N_NODES, :] = h0i
    aa = aa_ref[...]
    elr_u = lax.dot_general(aa, h0u, (((1,), (1,)), ((), ())),
                            preferred_element_type=jnp.float32)
    elr_i = lax.dot_general(aa, h0i, (((1,), (1,)), ((), ())),
                            preferred_element_type=jnp.float32)
    elr_ref[:, 0:N_USER] = elr_u
    elr_ref[:, N_USER:N_NODES] = elr_i
    mx = (jnp.maximum(jnp.max(elr_u[0]), jnp.max(elr_i[0]))
          + jnp.maximum(jnp.max(elr_u[1]), jnp.max(elr_i[1])))
    c_ref[...] = jnp.where(mx > 0, mx, 0.2 * mx).reshape(1, 1)


def _tc_pre(feat_user, feat_item, w_u, b_u, w_i, b_i, w_g, attn2):
    return pl.pallas_call(
        _tc_pre_body,
        out_shape=[
            jax.ShapeDtypeStruct((N_NODES, 128), jnp.float32),
            jax.ShapeDtypeStruct((2, N_NODES), jnp.float32),
            jax.ShapeDtypeStruct((1, 1), jnp.float32),
        ],
    )(feat_user, feat_item, w_u, b_u, w_i, b_i, w_g, attn2)


@functools.partial(
    pl.kernel,
    out_type=[jax.ShapeDtypeStruct((NC, NDR, 128), jnp.float32),
              jax.ShapeDtypeStruct((E_PAD,), jnp.float32)],
    mesh=_mesh,
    compiler_params=pltpu.CompilerParams(needs_layout_passes=False),
    scratch_types=[
        pltpu.VMEM((N_NODES,), jnp.float32),
        pltpu.VMEM((N_NODES,), jnp.float32),
        pltpu.VMEM((NDR, 128), jnp.float32),
        pltpu.VMEM((EPW,), jnp.int32),
        pltpu.VMEM((EPW,), jnp.int32),
        pltpu.VMEM((EPW,), jnp.float32),
        pltpu.VMEM((L,), jnp.float32),
        pltpu.VMEM((NDR,), jnp.int32),
        pltpu.VMEM_SHARED((NDR, 128), jnp.float32),
    ],
)
def _sc_a(src_hbm, dst_hbm, el_hbm, er_hbm, c_hbm, den_out, ee_out,
          el_v, er_v, den_v, src_v, dst_v, ee_v, c_v, iota_v, shared_den):
    c = lax.axis_index("c").astype(jnp.int32)
    s = lax.axis_index("s").astype(jnp.int32)
    wid = s * jnp.int32(NC) + c
    base = wid * jnp.int32(EPW)
    pltpu.sync_copy(el_hbm, el_v)
    pltpu.sync_copy(er_hbm, er_v)
    pltpu.sync_copy(c_hbm, c_v)
    pltpu.sync_copy(src_hbm.at[pl.ds(base, EPW)], src_v)
    pltpu.sync_copy(dst_hbm.at[pl.ds(base, EPW)], dst_v)

    zero = jnp.zeros((L,), jnp.float32)
    iota = lax.iota(jnp.int32, L)

    def zbody(i, carry):
        for k in range(8):
            den_v[i, pl.ds(k * L, L)] = zero
        return carry
    _fori(0, NDR, zbody)

    def ibody(i, carry):
        iota_v[pl.ds(i * jnp.int32(L), L)] = iota + i * jnp.int32(L)
        return carry
    _fori(0, NDR // L, ibody)

    @pl.when(s == 0)
    def _():
        pltpu.sync_copy(den_v, shared_den)
    plsc.subcore_barrier()

    cvec = c_v[...]

    def ebody(i, carry):
        off = i * jnp.int32(L)
        sv = src_v[pl.ds(off, L)]
        dv = dst_v[pl.ds(off, L)]
        e = plsc.load_gather(el_v, [sv]) + plsc.load_gather(er_v, [dv])
        e = jnp.where(e > 0, e, 0.2 * e)
        ee = jnp.exp(e - cvec)
        gidx = (base + off) + iota
        ee = jnp.where(gidx < jnp.int32(E_SL), ee, 0.0)
        ee_v[pl.ds(off, L)] = ee
        plsc.addupdate_scatter(
            den_v, [lax.shift_right_logical(dv, jnp.int32(7)),
                    lax.bitwise_and(dv, jnp.int32(127))], ee)
        return carry
    _fori(0, EPW // L, ebody)

    pltpu.sync_copy(ee_v, ee_out.at[pl.ds(base, EPW)])
    pltpu.sync_copy(den_v, shared_den.at[iota_v], add=True)
    plsc.subcore_barrier()

    @pl.when(s == 0)
    def _():
        pltpu.sync_copy(shared_den, den_out.at[c])


@functools.partial(
    pl.kernel,
    out_type=jax.ShapeDtypeStruct((NC, ND_PAD, 128), jnp.float32),
    mesh=_mesh,
    compiler_params=pltpu.CompilerParams(needs_layout_passes=False),
    scratch_types=[
        pltpu.VMEM((CPS, G), jnp.int32),
        pltpu.VMEM((CPS, G), jnp.int32),
        pltpu.VMEM((CPS, G), jnp.float32),
        pltpu.VMEM((G, 128), jnp.float32),
        pltpu.VMEM_SHARED((ND_PAD, 128), jnp.float32),
    ],
)
def _sc_b(src3_hbm, dst3_hbm, ee3_hbm, h0_hbm,
          agg_out, src_v, dst_v, ee_v, rows_v, shared_out):
    c = lax.axis_index("c").astype(jnp.int32)
    s = lax.axis_index("s").astype(jnp.int32)
    wid = s * jnp.int32(NC) + c

    zero = jnp.zeros((L,), jnp.float32)

    def zbody(i, carry):
        for k in range(8):
            rows_v[i, pl.ds(k * L, L)] = zero
        return carry
    _fori(0, G, zbody)
    row0 = s * jnp.int32(ROWS_PS)
    for k in range(ROWS_PS // G):
        pltpu.sync_copy(rows_v, shared_out.at[pl.ds(row0 + k * G, G)])
    plsc.subcore_barrier()

    def seg_body(seg, carry0):
        ch0 = seg * jnp.int32(CPS)
        pltpu.sync_copy(src3_hbm.at[wid, pl.ds(ch0, CPS)], src_v)
        pltpu.sync_copy(dst3_hbm.at[wid, pl.ds(ch0, CPS)], dst_v)
        pltpu.sync_copy(ee3_hbm.at[wid, pl.ds(ch0, CPS)], ee_v)

        def chunk(ch, carry):
            pltpu.sync_copy(h0_hbm.at[src_v.at[ch]], rows_v)

            def sbody(j, carry2):
                a16 = plsc.load_gather(
                    ee_v, [jnp.broadcast_to(ch, (L,)).astype(jnp.int32),
                           jnp.broadcast_to(j, (L,)).astype(jnp.int32)])
                for k in range(8):
                    sl = pl.ds(k * L, L)
                    rows_v[j, sl] = rows_v[j, sl] * a16
                return carry2
            _fori(0, G, sbody)

            pltpu.sync_copy(rows_v, shared_out.at[dst_v.at[ch]], add=True)
            return carry
        _fori(0, CPS, chunk)
        return carry0
    _fori(0, NSEG, seg_body)

    plsc.subcore_barrier()
    for k in range(ROWS_PS // G):
        r = row0 + k * G
        pltpu.sync_copy(shared_out.at[pl.ds(r, G)],
                        agg_out.at[c, pl.ds(r, G)])


def _tc_mid_body(p_ref, den_ref, bg_ref, wp_ref, bp_ref, h_ref, out_ref):
    sfull = ((p_ref[0, 0:N_NODES, :] + p_ref[1, 0:N_NODES, :])
             / den_ref[...] + bg_ref[...])
    h = jnp.where(sfull > 0, sfull, jnp.exp(jnp.minimum(sfull, 0.0)) - 1.0)
    h_ref[...] = h
    out_ref[...] = (jnp.dot(h[0:N_USER, :], wp_ref[...],
                            preferred_element_type=jnp.float32) + bp_ref[...])


def _tc_mid(agg2, den, b_g, w_p, b_p):
    return pl.pallas_call(
        _tc_mid_body,
        out_shape=[
            jax.ShapeDtypeStruct((N_NODES, 128), jnp.float32),
            jax.ShapeDtypeStruct((N_USER, 64), jnp.float32),
        ],
    )(agg2, den, b_g, w_p, b_p)


@functools.partial(
    pl.kernel,
    out_type=jax.ShapeDtypeStruct((B_SCORE, 128), jnp.float32),
    mesh=_mesh,
    compiler_params=pltpu.CompilerParams(needs_layout_passes=False),
    scratch_types=[
        pltpu.VMEM((NCHC, GC), jnp.int32),
        pltpu.VMEM((GC, 128), jnp.float32),
    ],
)
def _sc_c(idx3_hbm, h_hbm, rows_out, idx_v, rows_v):
    c = lax.axis_index("c").astype(jnp.int32)
    s = lax.axis_index("s").astype(jnp.int32)
    wid = s * jnp.int32(NC) + c
    pltpu.sync_copy(idx3_hbm.at[wid], idx_v)

    def chunk(ch, carry):
        pltpu.sync_copy(h_hbm.at[idx_v.at[ch]], rows_v)
        pltpu.sync_copy(rows_v,
                        rows_out.at[pl.ds(wid * jnp.int32(BPW) + ch * jnp.int32(GC), GC)])
        return carry
    _fori(0, NCHC, chunk)


def _tc_loss_body(rows_ref, at_ref, loss_ref):
    ps = jnp.sum(rows_ref[0] * rows_ref[1], axis=1)
    ns = jnp.sum(rows_ref[2] * rows_ref[3], axis=1)
    k_t = jnp.minimum(jnp.float32(0.8), at_ref[0, 0])

    def part(sc):
        l = jnp.minimum(sc, 0.0) - jnp.log(1.0 + jnp.exp(-jnp.abs(sc)))
        hold = k_t * jnp.max(l)
        l = jnp.where(l > hold, 0.0, l)
        return -jnp.sum(l)

    loss_ref[...] = (part(ps) + part(ns)).reshape(1, 1)


def _tc_loss(rows4, alpha_t):
    return pl.pallas_call(
        _tc_loss_body,
        out_shape=jax.ShapeDtypeStruct((1, 1), jnp.float32),
    )(rows4, alpha_t)


def kernel(feat_user, feat_item, edge_index, pos_src, pos_dst, neg_src,
           neg_dst, alpha_T, W_user, b_user, W_item, b_item, W_gat,
           attn_l, attn_r, b_gat, W_pred, b_pred):
    f32 = jnp.float32
    i32 = jnp.int32
    loop = jnp.arange(N_NODES, dtype=i32)
    pad = jnp.zeros((E_PAD - E_SL,), dtype=i32)
    src = jnp.concatenate([edge_index[0].astype(i32), loop, pad])
    dst = jnp.concatenate([edge_index[1].astype(i32), loop, pad])
    attn2 = jnp.stack([attn_l, attn_r], axis=0).astype(f32)

    h0, elr, c11 = _tc_pre(
        feat_user.astype(f32), feat_item.astype(f32),
        W_user.astype(f32), b_user.reshape(1, -1).astype(f32),
        W_item.astype(f32), b_item.reshape(1, -1).astype(f32),
        W_gat.astype(f32), attn2)
    el = elr[0]
    er = elr[1]
    c16 = jnp.broadcast_to(c11.reshape(()), (L,))

    den2, ee = _sc_a(src, dst, el, er, c16)

    src3 = src.reshape(NW, NCH, G)
    dst3 = dst.reshape(NW, NCH, G)
    ee3 = ee.reshape(NW, NCH, G)
    agg2 = _sc_b(src3, dst3, ee3, h0)

    den = (den2[0] + den2[1]).reshape(ND_PAD)[:N_NODES].reshape(N_NODES, 1)

    h, out_pred = _tc_mid(agg2, den, b_gat.reshape(1, -1).astype(f32),
                          W_pred.astype(f32), b_pred.reshape(1, -1).astype(f32))

    idx3 = jnp.concatenate([pos_src, pos_dst, neg_src, neg_dst]
                           ).astype(i32).reshape(NW, NCHC, GC)
    rows = _sc_c(idx3, h)
    rows4 = rows.reshape(4, P, 128)

    loss11 = _tc_loss(rows4, alpha_T.reshape(1, 1).astype(f32))
    return (loss11[0, 0], out_pred)


# speedup vs baseline: 12.4569x; 1.0904x over previous
import functools

import jax
import jax.numpy as jnp
from jax import lax
from jax.experimental import pallas as pl
from jax.experimental.pallas import tpu as pltpu
from jax.experimental.pallas import tpu_sc as plsc

N_USER = 5000
N_ITEM = 5000
N_NODES = N_USER + N_ITEM
E_RAW = 320000
E_SL = E_RAW + N_NODES
P = 8192

NC = 2
NS = 16
NW = NC * NS
L = 16

E_PAD = 344064
EPW = E_PAD // NW
G = 64
NCH = EPW // G
NSEG = 3
CPS = NCH // NSEG
NDV = N_NODES // L
ND_PAD = 10240
NDR = ND_PAD // 128
ROWS_PT = N_NODES // NW
ROWS_PS = ND_PAD // NS
ZB = 128
B_SCORE = 4 * P
BPW = B_SCORE // NW
GC = 128
NCHC = BPW // GC

_mesh = plsc.VectorSubcoreMesh(core_axis_name="c", subcore_axis_name="s")


def _fori(lo, hi, body):
    lax.fori_loop(jnp.int32(lo), jnp.int32(hi), body, 0)


def _tc_pre_body(fu_ref, fi_ref, wu_ref, bu_ref, wi_ref, bi_ref, wg_ref,
                 aa_ref, h0_ref, elr_ref, c_ref):
    wg = wg_ref[...]
    m_u = jnp.dot(wu_ref[...], wg, preferred_element_type=jnp.float32)
    m_i = jnp.dot(wi_ref[...], wg, preferred_element_type=jnp.float32)
    cb_u = jnp.dot(bu_ref[...], wg, preferred_element_type=jnp.float32)
    cb_i = jnp.dot(bi_ref[...], wg, preferred_element_type=jnp.float32)
    h0u = jnp.dot(fu_ref[...], m_u, preferred_element_type=jnp.float32) + cb_u
    h0i = jnp.dot(fi_ref[...], m_i, preferred_element_type=jnp.float32) + cb_i
    h0_ref[0:N_USER, :] = h0u
    h0_ref[N_USER:N_NODES, :] = h0i
    aa = aa_ref[...]
    elr_u = lax.dot_general(aa, h0u, (((1,), (1,)), ((), ())),
                            preferred_element_type=jnp.float32)
    elr_i = lax.dot_general(aa, h0i, (((1,), (1,)), ((), ())),
                            preferred_element_type=jnp.float32)
    elr_ref[:, 0:N_USER] = elr_u
    elr_ref[:, N_USER:N_NODES] = elr_i
    mx = (jnp.maximum(jnp.max(elr_u[0]), jnp.max(elr_i[0]))
          + jnp.maximum(jnp.max(elr_u[1]), jnp.max(elr_i[1])))
    c_ref[...] = jnp.where(mx > 0, mx, 0.2 * mx).reshape(1, 1)


def _tc_pre(feat_user, feat_item, w_u, b_u, w_i, b_i, w_g, attn2):
    return pl.pallas_call(
        _tc_pre_body,
        out_shape=[
            jax.ShapeDtypeStruct((N_NODES, 128), jnp.float32),
            jax.ShapeDtypeStruct((2, N_NODES), jnp.float32),
            jax.ShapeDtypeStruct((1, 1), jnp.float32),
        ],
    )(feat_user, feat_item, w_u, b_u, w_i, b_i, w_g, attn2)


@functools.partial(
    pl.kernel,
    out_type=[jax.ShapeDtypeStruct((NC, NDR, 128), jnp.float32),
              jax.ShapeDtypeStruct((E_PAD,), jnp.float32)],
    mesh=_mesh,
    compiler_params=pltpu.CompilerParams(needs_layout_passes=False),
    scratch_types=[
        pltpu.VMEM((N_NODES,), jnp.float32),
        pltpu.VMEM((N_NODES,), jnp.float32),
        pltpu.VMEM((NDR, 128), jnp.float32),
        pltpu.VMEM((EPW,), jnp.int32),
        pltpu.VMEM((EPW,), jnp.int32),
        pltpu.VMEM((EPW,), jnp.float32),
        pltpu.VMEM((L,), jnp.float32),
        pltpu.VMEM((NDR,), jnp.int32),
        pltpu.VMEM_SHARED((NDR, 128), jnp.float32),
    ],
)
def _sc_a(src_hbm, dst_hbm, el_hbm, er_hbm, c_hbm, den_out, ee_out,
          el_v, er_v, den_v, src_v, dst_v, ee_v, c_v, iota_v, shared_den):
    c = lax.axis_index("c").astype(jnp.int32)
    s = lax.axis_index("s").astype(jnp.int32)
    wid = s * jnp.int32(NC) + c
    base = wid * jnp.int32(EPW)
    pltpu.sync_copy(el_hbm, el_v)
    pltpu.sync_copy(er_hbm, er_v)
    pltpu.sync_copy(c_hbm, c_v)
    pltpu.sync_copy(src_hbm.at[pl.ds(base, EPW)], src_v)
    pltpu.sync_copy(dst_hbm.at[pl.ds(base, EPW)], dst_v)

    zero = jnp.zeros((L,), jnp.float32)
    iota = lax.iota(jnp.int32, L)

    def zbody(i, carry):
        for k in range(8):
            den_v[i, pl.ds(k * L, L)] = zero
        return carry
    _fori(0, NDR, zbody)

    def ibody(i, carry):
        iota_v[pl.ds(i * jnp.int32(L), L)] = iota + i * jnp.int32(L)
        return carry
    _fori(0, NDR // L, ibody)

    @pl.when(s == 0)
    def _():
        pltpu.sync_copy(den_v, shared_den)
    plsc.subcore_barrier()

    cvec = c_v[...]

    def ebody(i, carry):
        off = i * jnp.int32(L)
        sv = src_v[pl.ds(off, L)]
        dv = dst_v[pl.ds(off, L)]
        e = plsc.load_gather(el_v, [sv]) + plsc.load_gather(er_v, [dv])
        e = jnp.where(e > 0, e, 0.2 * e)
        ee = jnp.exp(e - cvec)
        gidx = (base + off) + iota
        ee = jnp.where(gidx < jnp.int32(E_SL), ee, 0.0)
        ee_v[pl.ds(off, L)] = ee
        plsc.addupdate_scatter(
            den_v, [lax.shift_right_logical(dv, jnp.int32(7)),
                    lax.bitwise_and(dv, jnp.int32(127))], ee)
        return carry
    _fori(0, EPW // L, ebody)

    pltpu.sync_copy(ee_v, ee_out.at[pl.ds(base, EPW)])
    pltpu.sync_copy(den_v, shared_den.at[iota_v], add=True)
    plsc.subcore_barrier()

    @pl.when(s == 0)
    def _():
        pltpu.sync_copy(shared_den, den_out.at[c])


@functools.partial(
    pl.kernel,
    out_type=jax.ShapeDtypeStruct((NC, ND_PAD, 128), jnp.float32),
    mesh=_mesh,
    compiler_params=pltpu.CompilerParams(needs_layout_passes=False),
    scratch_types=[
        pltpu.VMEM((CPS, G), jnp.int32),
        pltpu.VMEM((CPS, G), jnp.int32),
        pltpu.VMEM((CPS, G), jnp.float32),
        pltpu.VMEM((G, 128), jnp.float32),
        pltpu.VMEM((G, 128), jnp.float32),
        pltpu.SemaphoreType.DMA,
        pltpu.SemaphoreType.DMA,
        pltpu.SemaphoreType.DMA,
        pltpu.SemaphoreType.DMA,
        pltpu.VMEM_SHARED((ND_PAD, 128), jnp.float32),
    ],
)
def _sc_b(src3_hbm, dst3_hbm, ee3_hbm, h0_hbm,
          agg_out, src_v, dst_v, ee_v, rows_a, rows_b,
          gsa, gsb, ssa, ssb, shared_out):
    c = lax.axis_index("c").astype(jnp.int32)
    s = lax.axis_index("s").astype(jnp.int32)
    wid = s * jnp.int32(NC) + c

    zero = jnp.zeros((L,), jnp.float32)

    def zbody(i, carry):
        for k in range(8):
            rows_a[i, pl.ds(k * L, L)] = zero
        return carry
    _fori(0, G, zbody)
    row0 = s * jnp.int32(ROWS_PS)
    for k in range(ROWS_PS // G):
        pltpu.sync_copy(rows_a, shared_out.at[pl.ds(row0 + k * G, G)])
    plsc.subcore_barrier()

    def scale(rows_v, ch):
        def sbody(j, carry2):
            a16 = plsc.load_gather(
                ee_v, [jnp.broadcast_to(ch, (L,)).astype(jnp.int32),
                       jnp.broadcast_to(j, (L,)).astype(jnp.int32)])
            for k in range(8):
                sl = pl.ds(k * L, L)
                rows_v[j, sl] = rows_v[j, sl] * a16
            return carry2
        _fori(0, G, sbody)

    NP = CPS // 2

    def seg_body(seg, carry0):
        ch0 = seg * jnp.int32(CPS)
        pltpu.sync_copy(src3_hbm.at[wid, pl.ds(ch0, CPS)], src_v)
        pltpu.sync_copy(dst3_hbm.at[wid, pl.ds(ch0, CPS)], dst_v)
        pltpu.sync_copy(ee3_hbm.at[wid, pl.ds(ch0, CPS)], ee_v)

        pltpu.async_copy(h0_hbm.at[src_v.at[jnp.int32(0)]], rows_a, gsa)

        def pbody(i, carry):
            cha = i * jnp.int32(2)
            chb = cha + jnp.int32(1)
            pltpu.make_async_copy(
                h0_hbm.at[src_v.at[cha]], rows_a, gsa).wait()

            @pl.when(i > 0)
            def _():
                pltpu.make_async_copy(
                    rows_b, shared_out.at[dst_v.at[chb]], ssb).wait()
            pltpu.async_copy(h0_hbm.at[src_v.at[chb]], rows_b, gsb)
            scale(rows_a, cha)
            pltpu.async_copy(rows_a, shared_out.at[dst_v.at[cha]], ssa,
                             add=True)
            pltpu.make_async_copy(
                h0_hbm.at[src_v.at[chb]], rows_b, gsb).wait()
            scale(rows_b, chb)
            pltpu.make_async_copy(
                rows_a, shared_out.at[dst_v.at[cha]], ssa).wait()

            @pl.when(i < NP - 1)
            def _():
                pltpu.async_copy(
                    h0_hbm.at[src_v.at[cha + jnp.int32(2)]], rows_a, gsa)
            pltpu.async_copy(rows_b, shared_out.at[dst_v.at[chb]], ssb,
                             add=True)
            return carry
        _fori(0, NP, pbody)
        pltpu.make_async_copy(
            rows_b, shared_out.at[dst_v.at[jnp.int32(CPS - 1)]], ssb).wait()
        return carry0
    _fori(0, NSEG, seg_body)

    plsc.subcore_barrier()
    for k in range(ROWS_PS // G):
        r = row0 + k * G
        pltpu.sync_copy(shared_out.at[pl.ds(r, G)],
                        agg_out.at[c, pl.ds(r, G)])


def _tc_mid_body(p_ref, den_ref, bg_ref, wp_ref, bp_ref, h_ref, out_ref):
    sfull = ((p_ref[0, 0:N_NODES, :] + p_ref[1, 0:N_NODES, :])
             / den_ref[...] + bg_ref[...])
    h = jnp.where(sfull > 0, sfull, jnp.exp(jnp.minimum(sfull, 0.0)) - 1.0)
    h_ref[...] = h
    out_ref[...] = (jnp.dot(h[0:N_USER, :], wp_ref[...],
                            preferred_element_type=jnp.float32) + bp_ref[...])


def _tc_mid(agg2, den, b_g, w_p, b_p):
    return pl.pallas_call(
        _tc_mid_body,
        out_shape=[
            jax.ShapeDtypeStruct((N_NODES, 128), jnp.float32),
            jax.ShapeDtypeStruct((N_USER, 64), jnp.float32),
        ],
    )(agg2, den, b_g, w_p, b_p)


@functools.partial(
    pl.kernel,
    out_type=jax.ShapeDtypeStruct((B_SCORE, 128), jnp.float32),
    mesh=_mesh,
    compiler_params=pltpu.CompilerParams(needs_layout_passes=False),
    scratch_types=[
        pltpu.VMEM((NCHC, GC), jnp.int32),
        pltpu.VMEM((GC, 128), jnp.float32),
    ],
)
def _sc_c(idx3_hbm, h_hbm, rows_out, idx_v, rows_v):
    c = lax.axis_index("c").astype(jnp.int32)
    s = lax.axis_index("s").astype(jnp.int32)
    wid = s * jnp.int32(NC) + c
    pltpu.sync_copy(idx3_hbm.at[wid], idx_v)

    def chunk(ch, carry):
        pltpu.sync_copy(h_hbm.at[idx_v.at[ch]], rows_v)
        pltpu.sync_copy(rows_v,
                        rows_out.at[pl.ds(wid * jnp.int32(BPW) + ch * jnp.int32(GC), GC)])
        return carry
    _fori(0, NCHC, chunk)


def _tc_loss_body(rows_ref, at_ref, loss_ref):
    ps = jnp.sum(rows_ref[0] * rows_ref[1], axis=1)
    ns = jnp.sum(rows_ref[2] * rows_ref[3], axis=1)
    k_t = jnp.minimum(jnp.float32(0.8), at_ref[0, 0])

    def part(sc):
        l = jnp.minimum(sc, 0.0) - jnp.log(1.0 + jnp.exp(-jnp.abs(sc)))
        hold = k_t * jnp.max(l)
        l = jnp.where(l > hold, 0.0, l)
        return -jnp.sum(l)

    loss_ref[...] = (part(ps) + part(ns)).reshape(1, 1)


def _tc_loss(rows4, alpha_t):
    return pl.pallas_call(
        _tc_loss_body,
        out_shape=jax.ShapeDtypeStruct((1, 1), jnp.float32),
    )(rows4, alpha_t)


def kernel(feat_user, feat_item, edge_index, pos_src, pos_dst, neg_src,
           neg_dst, alpha_T, W_user, b_user, W_item, b_item, W_gat,
           attn_l, attn_r, b_gat, W_pred, b_pred):
    f32 = jnp.float32
    i32 = jnp.int32
    loop = jnp.arange(N_NODES, dtype=i32)
    pad = jnp.zeros((E_PAD - E_SL,), dtype=i32)
    src = jnp.concatenate([edge_index[0].astype(i32), loop, pad])
    dst = jnp.concatenate([edge_index[1].astype(i32), loop, pad])
    attn2 = jnp.stack([attn_l, attn_r], axis=0).astype(f32)

    h0, elr, c11 = _tc_pre(
        feat_user.astype(f32), feat_item.astype(f32),
        W_user.astype(f32), b_user.reshape(1, -1).astype(f32),
        W_item.astype(f32), b_item.reshape(1, -1).astype(f32),
        W_gat.astype(f32), attn2)
    el = elr[0]
    er = elr[1]
    c16 = jnp.broadcast_to(c11.reshape(()), (L,))

    den2, ee = _sc_a(src, dst, el, er, c16)

    src3 = src.reshape(NW, NCH, G)
    dst3 = dst.reshape(NW, NCH, G)
    ee3 = ee.reshape(NW, NCH, G)
    agg2 = _sc_b(src3, dst3, ee3, h0)

    den = (den2[0] + den2[1]).reshape(ND_PAD)[:N_NODES].reshape(N_NODES, 1)

    h, out_pred = _tc_mid(agg2, den, b_gat.reshape(1, -1).astype(f32),
                          W_pred.astype(f32), b_pred.reshape(1, -1).astype(f32))

    idx3 = jnp.concatenate([pos_src, pos_dst, neg_src, neg_dst]
                           ).astype(i32).reshape(NW, NCHC, GC)
    rows = _sc_c(idx3, h)
    rows4 = rows.reshape(4, P, 128)

    loss11 = _tc_loss(rows4, alpha_T.reshape(1, 1).astype(f32))
    return (loss11[0, 0], out_pred)


# speedup vs baseline: 12.5055x; 1.0039x over previous
import functools

import jax
import jax.numpy as jnp
from jax import lax
from jax.experimental import pallas as pl
from jax.experimental.pallas import tpu as pltpu
from jax.experimental.pallas import tpu_sc as plsc

N_USER = 5000
N_ITEM = 5000
N_NODES = N_USER + N_ITEM
E_RAW = 320000
E_SL = E_RAW + N_NODES
P = 8192

NC = 2
NS = 16
NW = NC * NS
L = 16

E_PAD = 344064
EPW = E_PAD // NW
G = 64
F_PT = 4
ECH = 4096
NCHB = E_PAD // ECH
NDV = N_NODES // L
ND_PAD = 10240
NDR = ND_PAD // 128
ROWS_PT = N_NODES // NW
ROWS_PS = ND_PAD // NS
ZB = 128
B_SCORE = 4 * P
BPW = B_SCORE // NW
GC = 128
NCHC = BPW // GC

_mesh = plsc.VectorSubcoreMesh(core_axis_name="c", subcore_axis_name="s")


def _fori(lo, hi, body):
    lax.fori_loop(jnp.int32(lo), jnp.int32(hi), body, 0)


def _tc_pre_body(fu_ref, fi_ref, wu_ref, bu_ref, wi_ref, bi_ref, wg_ref,
                 aa_ref, h0_ref, elr_ref, c_ref):
    wg = wg_ref[...]
    m_u = jnp.dot(wu_ref[...], wg, preferred_element_type=jnp.float32)
    m_i = jnp.dot(wi_ref[...], wg, preferred_element_type=jnp.float32)
    cb_u = jnp.dot(bu_ref[...], wg, preferred_element_type=jnp.float32)
    cb_i = jnp.dot(bi_ref[...], wg, preferred_element_type=jnp.float32)
    h0u = jnp.dot(fu_ref[...], m_u, preferred_element_type=jnp.float32) + cb_u
    h0i = jnp.dot(fi_ref[...], m_i, preferred_element_type=jnp.float32) + cb_i
    h0_ref[0:N_USER, :] = h0u
    h0_ref[N_USER:N_NODES, :] = h0i
    aa = aa_ref[...]
    elr_u = lax.dot_general(aa, h0u, (((1,), (1,)), ((), ())),
                            preferred_element_type=jnp.float32)
    elr_i = lax.dot_general(aa, h0i, (((1,), (1,)), ((), ())),
                            preferred_element_type=jnp.float32)
    elr_ref[:, 0:N_USER] = elr_u
    elr_ref[:, N_USER:N_NODES] = elr_i
    mx = (jnp.maximum(jnp.max(elr_u[0]), jnp.max(elr_i[0]))
          + jnp.maximum(jnp.max(elr_u[1]), jnp.max(elr_i[1])))
    c_ref[...] = jnp.where(mx > 0, mx, 0.2 * mx).reshape(1, 1)


def _tc_pre(feat_user, feat_item, w_u, b_u, w_i, b_i, w_g, attn2):
    return pl.pallas_call(
        _tc_pre_body,
        out_shape=[
            jax.ShapeDtypeStruct((N_NODES, 128), jnp.float32),
            jax.ShapeDtypeStruct((2, N_NODES), jnp.float32),
            jax.ShapeDtypeStruct((1, 1), jnp.float32),
        ],
    )(feat_user, feat_item, w_u, b_u, w_i, b_i, w_g, attn2)


@functools.partial(
    pl.kernel,
    out_type=[jax.ShapeDtypeStruct((NC, NDR, 128), jnp.float32),
              jax.ShapeDtypeStruct((E_PAD,), jnp.float32)],
    mesh=_mesh,
    compiler_params=pltpu.CompilerParams(needs_layout_passes=False),
    scratch_types=[
        pltpu.VMEM((N_NODES,), jnp.float32),
        pltpu.VMEM((N_NODES,), jnp.float32),
        pltpu.VMEM((NDR, 128), jnp.float32),
        pltpu.VMEM((EPW,), jnp.int32),
        pltpu.VMEM((EPW,), jnp.int32),
        pltpu.VMEM((EPW,), jnp.float32),
        pltpu.VMEM((L,), jnp.float32),
        pltpu.VMEM((NDR,), jnp.int32),
        pltpu.VMEM_SHARED((NDR, 128), jnp.float32),
    ],
)
def _sc_a(src_hbm, dst_hbm, el_hbm, er_hbm, c_hbm, den_out, ee_out,
          el_v, er_v, den_v, src_v, dst_v, ee_v, c_v, iota_v, shared_den):
    c = lax.axis_index("c").astype(jnp.int32)
    s = lax.axis_index("s").astype(jnp.int32)
    wid = s * jnp.int32(NC) + c
    base = wid * jnp.int32(EPW)
    pltpu.sync_copy(el_hbm, el_v)
    pltpu.sync_copy(er_hbm, er_v)
    pltpu.sync_copy(c_hbm, c_v)
    pltpu.sync_copy(src_hbm.at[pl.ds(base, EPW)], src_v)
    pltpu.sync_copy(dst_hbm.at[pl.ds(base, EPW)], dst_v)

    zero = jnp.zeros((L,), jnp.float32)
    iota = lax.iota(jnp.int32, L)

    def zbody(i, carry):
        for k in range(8):
            den_v[i, pl.ds(k * L, L)] = zero
        return carry
    _fori(0, NDR, zbody)

    def ibody(i, carry):
        iota_v[pl.ds(i * jnp.int32(L), L)] = iota + i * jnp.int32(L)
        return carry
    _fori(0, NDR // L, ibody)

    @pl.when(s == 0)
    def _():
        pltpu.sync_copy(den_v, shared_den)
    plsc.subcore_barrier()

    cvec = c_v[...]

    def ebody(i, carry):
        off = i * jnp.int32(L)
        sv = src_v[pl.ds(off, L)]
        dv = dst_v[pl.ds(off, L)]
        e = plsc.load_gather(el_v, [sv]) + plsc.load_gather(er_v, [dv])
        e = jnp.where(e > 0, e, 0.2 * e)
        ee = jnp.exp(e - cvec)
        gidx = (base + off) + iota
        ee = jnp.where(gidx < jnp.int32(E_SL), ee, 0.0)
        ee_v[pl.ds(off, L)] = ee
        plsc.addupdate_scatter(
            den_v, [lax.shift_right_logical(dv, jnp.int32(7)),
                    lax.bitwise_and(dv, jnp.int32(127))], ee)
        return carry
    _fori(0, EPW // L, ebody)

    pltpu.sync_copy(ee_v, ee_out.at[pl.ds(base, EPW)])
    pltpu.sync_copy(den_v, shared_den.at[iota_v], add=True)
    plsc.subcore_barrier()

    @pl.when(s == 0)
    def _():
        pltpu.sync_copy(shared_den, den_out.at[c])


@functools.partial(
    pl.kernel,
    out_type=jax.ShapeDtypeStruct((NW, F_PT, ND_PAD), jnp.float32),
    mesh=_mesh,
    compiler_params=pltpu.CompilerParams(needs_layout_passes=False),
    scratch_types=[
        pltpu.VMEM((F_PT, ND_PAD), jnp.float32),
        pltpu.VMEM((F_PT, ND_PAD), jnp.float32),
        pltpu.VMEM((ECH,), jnp.int32),
        pltpu.VMEM((ECH,), jnp.float32),
        pltpu.VMEM((ECH,), jnp.int32),
        pltpu.VMEM((ECH,), jnp.float32),
        pltpu.SemaphoreType.DMA,
        pltpu.SemaphoreType.DMA,
    ],
)
def _sc_b(pk_hbm, ee_hbm, h0t_hbm,
          agg_out, h0t_v, acc_v, pk_a, ee_a, pk_b, ee_b, sa, sb):
    c = lax.axis_index("c").astype(jnp.int32)
    s = lax.axis_index("s").astype(jnp.int32)
    wid = s * jnp.int32(NC) + c
    pltpu.sync_copy(h0t_hbm.at[wid], h0t_v)

    zero = jnp.zeros((L,), jnp.float32)

    def zbody(i, carry):
        for f in range(F_PT):
            acc_v[f, pl.ds(i * jnp.int32(L), L)] = zero
        return carry
    _fori(0, ND_PAD // L, zbody)

    fvec = [jnp.full((L,), f, jnp.int32) for f in range(F_PT)]

    def process(pk_v, ee_v):
        def ib(i, carry):
            off = i * jnp.int32(2 * L)
            for u in range(2):
                o = off + u * L
                pk = pk_v[pl.ds(o, L)]
                ee = ee_v[pl.ds(o, L)]
                sv = lax.shift_right_logical(pk, jnp.int32(14))
                dv = lax.bitwise_and(pk, jnp.int32(16383))
                for f in range(F_PT):
                    hv = plsc.load_gather(h0t_v, [fvec[f], sv])
                    plsc.addupdate_scatter(acc_v, [fvec[f], dv], hv * ee)
            return carry
        _fori(0, ECH // (2 * L), ib)

    def stage(ch, pk_v, ee_v, sem):
        e0 = ch * jnp.int32(ECH)
        pltpu.async_copy(pk_hbm.at[pl.ds(e0, ECH)], pk_v, sem)
        pltpu.async_copy(ee_hbm.at[pl.ds(e0, ECH)], ee_v, sem)

    def swait(pk_v, ee_v, sem):
        pltpu.make_async_copy(pk_hbm.at[pl.ds(0, ECH)], pk_v, sem).wait()
        pltpu.make_async_copy(ee_hbm.at[pl.ds(0, ECH)], ee_v, sem).wait()

    stage(jnp.int32(0), pk_a, ee_a, sa)
    NP = NCHB // 2

    def pbody(i, carry):
        cha = i * jnp.int32(2)
        chb = cha + jnp.int32(1)
        swait(pk_a, ee_a, sa)
        stage(chb, pk_b, ee_b, sb)
        process(pk_a, ee_a)
        swait(pk_b, ee_b, sb)

        @pl.when(i < NP - 1)
        def _():
            stage(cha + jnp.int32(2), pk_a, ee_a, sa)
        process(pk_b, ee_b)
        return carry
    _fori(0, NP, pbody)

    pltpu.sync_copy(acc_v, agg_out.at[wid])


def _tc_mid_body(p_ref, den_ref, bg_ref, wp_ref, bp_ref, h_ref, out_ref):
    sfull = p_ref[...] / den_ref[...] + bg_ref[...]
    h = jnp.where(sfull > 0, sfull, jnp.exp(jnp.minimum(sfull, 0.0)) - 1.0)
    h_ref[...] = h
    out_ref[...] = (jnp.dot(h[0:N_USER, :], wp_ref[...],
                            preferred_element_type=jnp.float32) + bp_ref[...])


def _tc_mid(agg2, den, b_g, w_p, b_p):
    return pl.pallas_call(
        _tc_mid_body,
        out_shape=[
            jax.ShapeDtypeStruct((N_NODES, 128), jnp.float32),
            jax.ShapeDtypeStruct((N_USER, 64), jnp.float32),
        ],
    )(agg2, den, b_g, w_p, b_p)


@functools.partial(
    pl.kernel,
    out_type=jax.ShapeDtypeStruct((B_SCORE, 128), jnp.float32),
    mesh=_mesh,
    compiler_params=pltpu.CompilerParams(needs_layout_passes=False),
    scratch_types=[
        pltpu.VMEM((NCHC, GC), jnp.int32),
        pltpu.VMEM((GC, 128), jnp.float32),
    ],
)
def _sc_c(idx3_hbm, h_hbm, rows_out, idx_v, rows_v):
    c = lax.axis_index("c").astype(jnp.int32)
    s = lax.axis_index("s").astype(jnp.int32)
    wid = s * jnp.int32(NC) + c
    pltpu.sync_copy(idx3_hbm.at[wid], idx_v)

    def chunk(ch, carry):
        pltpu.sync_copy(h_hbm.at[idx_v.at[ch]], rows_v)
        pltpu.sync_copy(rows_v,
                        rows_out.at[pl.ds(wid * jnp.int32(BPW) + ch * jnp.int32(GC), GC)])
        return carry
    _fori(0, NCHC, chunk)


def _tc_loss_body(rows_ref, at_ref, loss_ref):
    ps = jnp.sum(rows_ref[0] * rows_ref[1], axis=1)
    ns = jnp.sum(rows_ref[2] * rows_ref[3], axis=1)
    k_t = jnp.minimum(jnp.float32(0.8), at_ref[0, 0])

    def part(sc):
        l = jnp.minimum(sc, 0.0) - jnp.log(1.0 + jnp.exp(-jnp.abs(sc)))
        hold = k_t * jnp.max(l)
        l = jnp.where(l > hold, 0.0, l)
        return -jnp.sum(l)

    loss_ref[...] = (part(ps) + part(ns)).reshape(1, 1)


def _tc_loss(rows4, alpha_t):
    return pl.pallas_call(
        _tc_loss_body,
        out_shape=jax.ShapeDtypeStruct((1, 1), jnp.float32),
    )(rows4, alpha_t)


def kernel(feat_user, feat_item, edge_index, pos_src, pos_dst, neg_src,
           neg_dst, alpha_T, W_user, b_user, W_item, b_item, W_gat,
           attn_l, attn_r, b_gat, W_pred, b_pred):
    f32 = jnp.float32
    i32 = jnp.int32
    loop = jnp.arange(N_NODES, dtype=i32)
    pad = jnp.zeros((E_PAD - E_SL,), dtype=i32)
    src = jnp.concatenate([edge_index[0].astype(i32), loop, pad])
    dst = jnp.concatenate([edge_index[1].astype(i32), loop, pad])
    attn2 = jnp.stack([attn_l, attn_r], axis=0).astype(f32)

    h0, elr, c11 = _tc_pre(
        feat_user.astype(f32), feat_item.astype(f32),
        W_user.astype(f32), b_user.reshape(1, -1).astype(f32),
        W_item.astype(f32), b_item.reshape(1, -1).astype(f32),
        W_gat.astype(f32), attn2)
    el = elr[0]
    er = elr[1]
    c16 = jnp.broadcast_to(c11.reshape(()), (L,))

    den2, ee = _sc_a(src, dst, el, er, c16)

    packed = src * jnp.int32(16384) + dst
    h0p = jnp.zeros((ND_PAD, 128), jnp.float32).at[:N_NODES].set(h0)
    h0t = h0p.T.reshape(NW, F_PT, ND_PAD)
    agg4 = _sc_b(packed, ee, h0t)

    psum = agg4.reshape(128, ND_PAD)[:, :N_NODES].T
    den = (den2[0] + den2[1]).reshape(ND_PAD)[:N_NODES].reshape(N_NODES, 1)

    h, out_pred = _tc_mid(psum, den, b_gat.reshape(1, -1).astype(f32),
                          W_pred.astype(f32), b_pred.reshape(1, -1).astype(f32))

    idx3 = jnp.concatenate([pos_src, pos_dst, neg_src, neg_dst]
                           ).astype(i32).reshape(NW, NCHC, GC)
    rows = _sc_c(idx3, h)
    rows4 = rows.reshape(4, P, 128)

    loss11 = _tc_loss(rows4, alpha_T.reshape(1, 1).astype(f32))
    return (loss11[0, 0], out_pred)


# speedup vs baseline: 25.0560x; 2.0036x over previous
import functools

import jax
import jax.numpy as jnp
from jax import lax
from jax.experimental import pallas as pl
from jax.experimental.pallas import tpu as pltpu
from jax.experimental.pallas import tpu_sc as plsc

N_USER = 5000
N_ITEM = 5000
N_NODES = N_USER + N_ITEM
E_RAW = 320000
E_SL = E_RAW + N_NODES
P = 8192

NC = 2
NS = 16
NW = NC * NS
L = 16

E_PAD = 344064
EPW = E_PAD // NW
G = 64
F_PT = 4
ECH = 4096
NCHB = E_PAD // ECH
NDV = N_NODES // L
ND_PAD = 10240
NDR = ND_PAD // 128
ROWS_PT = N_NODES // NW
ROWS_PS = ND_PAD // NS
ZB = 128
B_SCORE = 4 * P
BPW = B_SCORE // NW
GC = 128
NCHC = BPW // GC

_mesh = plsc.VectorSubcoreMesh(core_axis_name="c", subcore_axis_name="s")


def _fori(lo, hi, body):
    lax.fori_loop(jnp.int32(lo), jnp.int32(hi), body, 0)


def _tc_pre_body(fu_ref, fi_ref, wu_ref, bu_ref, wi_ref, bi_ref, wg_ref,
                 aa_ref, h0_ref, elr_ref, c_ref):
    wg = wg_ref[...]
    m_u = jnp.dot(wu_ref[...], wg, preferred_element_type=jnp.float32)
    m_i = jnp.dot(wi_ref[...], wg, preferred_element_type=jnp.float32)
    cb_u = jnp.dot(bu_ref[...], wg, preferred_element_type=jnp.float32)
    cb_i = jnp.dot(bi_ref[...], wg, preferred_element_type=jnp.float32)
    h0u = jnp.dot(fu_ref[...], m_u, preferred_element_type=jnp.float32) + cb_u
    h0i = jnp.dot(fi_ref[...], m_i, preferred_element_type=jnp.float32) + cb_i
    h0_ref[0:N_USER, :] = h0u
    h0_ref[N_USER:N_NODES, :] = h0i
    aa = aa_ref[...]
    elr_u = lax.dot_general(aa, h0u, (((1,), (1,)), ((), ())),
                            preferred_element_type=jnp.float32)
    elr_i = lax.dot_general(aa, h0i, (((1,), (1,)), ((), ())),
                            preferred_element_type=jnp.float32)
    elr_ref[:, 0:N_USER] = elr_u
    elr_ref[:, N_USER:N_NODES] = elr_i
    mx = (jnp.maximum(jnp.max(elr_u[0]), jnp.max(elr_i[0]))
          + jnp.maximum(jnp.max(elr_u[1]), jnp.max(elr_i[1])))
    c_ref[...] = jnp.where(mx > 0, mx, 0.2 * mx).reshape(1, 1)


def _tc_pre(feat_user, feat_item, w_u, b_u, w_i, b_i, w_g, attn2):
    return pl.pallas_call(
        _tc_pre_body,
        out_shape=[
            jax.ShapeDtypeStruct((N_NODES, 128), jnp.float32),
            jax.ShapeDtypeStruct((2, N_NODES), jnp.float32),
            jax.ShapeDtypeStruct((1, 1), jnp.float32),
        ],
    )(feat_user, feat_item, w_u, b_u, w_i, b_i, w_g, attn2)


@functools.partial(
    pl.kernel,
    out_type=[jax.ShapeDtypeStruct((NC, NDR, 128), jnp.float32),
              jax.ShapeDtypeStruct((E_PAD,), jnp.float32)],
    mesh=_mesh,
    compiler_params=pltpu.CompilerParams(needs_layout_passes=False),
    scratch_types=[
        pltpu.VMEM((N_NODES,), jnp.float32),
        pltpu.VMEM((N_NODES,), jnp.float32),
        pltpu.VMEM((NDR, 128), jnp.float32),
        pltpu.VMEM((EPW,), jnp.int32),
        pltpu.VMEM((EPW,), jnp.int32),
        pltpu.VMEM((EPW,), jnp.float32),
        pltpu.VMEM((L,), jnp.float32),
        pltpu.VMEM((NDR,), jnp.int32),
        pltpu.VMEM_SHARED((NDR, 128), jnp.float32),
    ],
)
def _sc_a(src_hbm, dst_hbm, el_hbm, er_hbm, c_hbm, den_out, ee_out,
          el_v, er_v, den_v, src_v, dst_v, ee_v, c_v, iota_v, shared_den):
    c = lax.axis_index("c").astype(jnp.int32)
    s = lax.axis_index("s").astype(jnp.int32)
    wid = s * jnp.int32(NC) + c
    base = wid * jnp.int32(EPW)
    pltpu.sync_copy(el_hbm, el_v)
    pltpu.sync_copy(er_hbm, er_v)
    pltpu.sync_copy(c_hbm, c_v)
    pltpu.sync_copy(src_hbm.at[pl.ds(base, EPW)], src_v)
    pltpu.sync_copy(dst_hbm.at[pl.ds(base, EPW)], dst_v)

    zero = jnp.zeros((L,), jnp.float32)
    iota = lax.iota(jnp.int32, L)

    def zbody(i, carry):
        for k in range(8):
            den_v[i, pl.ds(k * L, L)] = zero
        return carry
    _fori(0, NDR, zbody)

    def ibody(i, carry):
        iota_v[pl.ds(i * jnp.int32(L), L)] = iota + i * jnp.int32(L)
        return carry
    _fori(0, NDR // L, ibody)

    @pl.when(s == 0)
    def _():
        pltpu.sync_copy(den_v, shared_den)
    plsc.subcore_barrier()

    cvec = c_v[...]

    def ebody(i, carry):
        off = i * jnp.int32(L)
        sv = src_v[pl.ds(off, L)]
        dv = dst_v[pl.ds(off, L)]
        e = plsc.load_gather(el_v, [sv]) + plsc.load_gather(er_v, [dv])
        e = jnp.where(e > 0, e, 0.2 * e)
        ee = jnp.exp(e - cvec)
        gidx = (base + off) + iota
        ee = jnp.where(gidx < jnp.int32(E_SL), ee, 0.0)
        ee_v[pl.ds(off, L)] = ee
        plsc.addupdate_scatter(
            den_v, [lax.shift_right_logical(dv, jnp.int32(7)),
                    lax.bitwise_and(dv, jnp.int32(127))], ee)
        return carry
    _fori(0, EPW // L, ebody)

    pltpu.sync_copy(ee_v, ee_out.at[pl.ds(base, EPW)])
    pltpu.sync_copy(den_v, shared_den.at[iota_v], add=True)
    plsc.subcore_barrier()

    @pl.when(s == 0)
    def _():
        pltpu.sync_copy(shared_den, den_out.at[c])


@functools.partial(
    pl.kernel,
    out_type=jax.ShapeDtypeStruct((NW, F_PT, ND_PAD), jnp.float32),
    mesh=_mesh,
    compiler_params=pltpu.CompilerParams(needs_layout_passes=False),
    scratch_types=[
        pltpu.VMEM((F_PT, ND_PAD), jnp.float32),
        pltpu.VMEM((F_PT, ND_PAD), jnp.float32),
        pltpu.VMEM((ECH,), jnp.int32),
        pltpu.VMEM((ECH,), jnp.float32),
        pltpu.VMEM((ECH,), jnp.int32),
        pltpu.VMEM((ECH,), jnp.float32),
        pltpu.SemaphoreType.DMA,
        pltpu.SemaphoreType.DMA,
    ],
)
def _sc_b(pk_hbm, ee_hbm, h0t_hbm,
          agg_out, h0t_v, acc_v, pk_a, ee_a, pk_b, ee_b, sa, sb):
    c = lax.axis_index("c").astype(jnp.int32)
    s = lax.axis_index("s").astype(jnp.int32)
    wid = s * jnp.int32(NC) + c
    pltpu.sync_copy(h0t_hbm.at[wid], h0t_v)

    zero = jnp.zeros((L,), jnp.float32)

    def zbody(i, carry):
        for f in range(F_PT):
            acc_v[f, pl.ds(i * jnp.int32(L), L)] = zero
        return carry
    _fori(0, ND_PAD // L, zbody)

    fvec = [jnp.full((L,), f, jnp.int32) for f in range(F_PT)]

    def process(pk_v, ee_v):
        UNR = 4

        def ib(i, carry):
            off = i * jnp.int32(UNR * L)
            svs, dvs, ees = [], [], []
            for u in range(UNR):
                o = off + u * L
                pk = pk_v[pl.ds(o, L)]
                ees.append(ee_v[pl.ds(o, L)])
                svs.append(lax.shift_right_logical(pk, jnp.int32(14)))
                dvs.append(lax.bitwise_and(pk, jnp.int32(16383)))
            hvs = [plsc.load_gather(h0t_v, [fvec[f], svs[u]])
                   for u in range(UNR) for f in range(F_PT)]
            ms = [hvs[u * F_PT + f] * ees[u]
                  for u in range(UNR) for f in range(F_PT)]
            for u in range(UNR):
                for f in range(F_PT):
                    plsc.addupdate_scatter(
                        acc_v, [fvec[f], dvs[u]], ms[u * F_PT + f])
            return carry
        _fori(0, ECH // (UNR * L), ib)

    def stage(ch, pk_v, ee_v, sem):
        e0 = ch * jnp.int32(ECH)
        pltpu.async_copy(pk_hbm.at[pl.ds(e0, ECH)], pk_v, sem)
        pltpu.async_copy(ee_hbm.at[pl.ds(e0, ECH)], ee_v, sem)

    def swait(pk_v, ee_v, sem):
        pltpu.make_async_copy(pk_hbm.at[pl.ds(0, ECH)], pk_v, sem).wait()
        pltpu.make_async_copy(ee_hbm.at[pl.ds(0, ECH)], ee_v, sem).wait()

    stage(jnp.int32(0), pk_a, ee_a, sa)
    NP = NCHB // 2

    def pbody(i, carry):
        cha = i * jnp.int32(2)
        chb = cha + jnp.int32(1)
        swait(pk_a, ee_a, sa)
        stage(chb, pk_b, ee_b, sb)
        process(pk_a, ee_a)
        swait(pk_b, ee_b, sb)

        @pl.when(i < NP - 1)
        def _():
            stage(cha + jnp.int32(2), pk_a, ee_a, sa)
        process(pk_b, ee_b)
        return carry
    _fori(0, NP, pbody)

    pltpu.sync_copy(acc_v, agg_out.at[wid])


def _tc_mid_body(p_ref, den_ref, bg_ref, wp_ref, bp_ref, h_ref, out_ref):
    sfull = p_ref[...] / den_ref[...] + bg_ref[...]
    h = jnp.where(sfull > 0, sfull, jnp.exp(jnp.minimum(sfull, 0.0)) - 1.0)
    h_ref[...] = h
    out_ref[...] = (jnp.dot(h[0:N_USER, :], wp_ref[...],
                            preferred_element_type=jnp.float32) + bp_ref[...])


def _tc_mid(agg2, den, b_g, w_p, b_p):
    return pl.pallas_call(
        _tc_mid_body,
        out_shape=[
            jax.ShapeDtypeStruct((N_NODES, 128), jnp.float32),
            jax.ShapeDtypeStruct((N_USER, 64), jnp.float32),
        ],
    )(agg2, den, b_g, w_p, b_p)


@functools.partial(
    pl.kernel,
    out_type=jax.ShapeDtypeStruct((B_SCORE, 128), jnp.float32),
    mesh=_mesh,
    compiler_params=pltpu.CompilerParams(needs_layout_passes=False),
    scratch_types=[
        pltpu.VMEM((NCHC, GC), jnp.int32),
        pltpu.VMEM((GC, 128), jnp.float32),
    ],
)
def _sc_c(idx3_hbm, h_hbm, rows_out, idx_v, rows_v):
    c = lax.axis_index("c").astype(jnp.int32)
    s = lax.axis_index("s").astype(jnp.int32)
    wid = s * jnp.int32(NC) + c
    pltpu.sync_copy(idx3_hbm.at[wid], idx_v)

    def chunk(ch, carry):
        pltpu.sync_copy(h_hbm.at[idx_v.at[ch]], rows_v)
        pltpu.sync_copy(rows_v,
                        rows_out.at[pl.ds(wid * jnp.int32(BPW) + ch * jnp.int32(GC), GC)])
        return carry
    _fori(0, NCHC, chunk)


def _tc_loss_body(rows_ref, at_ref, loss_ref):
    ps = jnp.sum(rows_ref[0] * rows_ref[1], axis=1)
    ns = jnp.sum(rows_ref[2] * rows_ref[3], axis=1)
    k_t = jnp.minimum(jnp.float32(0.8), at_ref[0, 0])

    def part(sc):
        l = jnp.minimum(sc, 0.0) - jnp.log(1.0 + jnp.exp(-jnp.abs(sc)))
        hold = k_t * jnp.max(l)
        l = jnp.where(l > hold, 0.0, l)
        return -jnp.sum(l)

    loss_ref[...] = (part(ps) + part(ns)).reshape(1, 1)


def _tc_loss(rows4, alpha_t):
    return pl.pallas_call(
        _tc_loss_body,
        out_shape=jax.ShapeDtypeStruct((1, 1), jnp.float32),
    )(rows4, alpha_t)


def kernel(feat_user, feat_item, edge_index, pos_src, pos_dst, neg_src,
           neg_dst, alpha_T, W_user, b_user, W_item, b_item, W_gat,
           attn_l, attn_r, b_gat, W_pred, b_pred):
    f32 = jnp.float32
    i32 = jnp.int32
    loop = jnp.arange(N_NODES, dtype=i32)
    pad = jnp.zeros((E_PAD - E_SL,), dtype=i32)
    src = jnp.concatenate([edge_index[0].astype(i32), loop, pad])
    dst = jnp.concatenate([edge_index[1].astype(i32), loop, pad])
    attn2 = jnp.stack([attn_l, attn_r], axis=0).astype(f32)

    h0, elr, c11 = _tc_pre(
        feat_user.astype(f32), feat_item.astype(f32),
        W_user.astype(f32), b_user.reshape(1, -1).astype(f32),
        W_item.astype(f32), b_item.reshape(1, -1).astype(f32),
        W_gat.astype(f32), attn2)
    el = elr[0]
    er = elr[1]
    c16 = jnp.broadcast_to(c11.reshape(()), (L,))

    den2, ee = _sc_a(src, dst, el, er, c16)

    packed = src * jnp.int32(16384) + dst
    h0p = jnp.zeros((ND_PAD, 128), jnp.float32).at[:N_NODES].set(h0)
    h0t = h0p.T.reshape(NW, F_PT, ND_PAD)
    agg4 = _sc_b(packed, ee, h0t)

    psum = agg4.reshape(128, ND_PAD)[:, :N_NODES].T
    den = (den2[0] + den2[1]).reshape(ND_PAD)[:N_NODES].reshape(N_NODES, 1)

    h, out_pred = _tc_mid(psum, den, b_gat.reshape(1, -1).astype(f32),
                          W_pred.astype(f32), b_pred.reshape(1, -1).astype(f32))

    idx3 = jnp.concatenate([pos_src, pos_dst, neg_src, neg_dst]
                           ).astype(i32).reshape(NW, NCHC, GC)
    rows = _sc_c(idx3, h)
    rows4 = rows.reshape(4, P, 128)

    loss11 = _tc_loss(rows4, alpha_T.reshape(1, 1).astype(f32))
    return (loss11[0, 0], out_pred)


# speedup vs baseline: 26.1297x; 1.0429x over previous
import functools

import jax
import jax.numpy as jnp
from jax import lax
from jax.experimental import pallas as pl
from jax.experimental.pallas import tpu as pltpu
from jax.experimental.pallas import tpu_sc as plsc

N_USER = 5000
N_ITEM = 5000
N_NODES = N_USER + N_ITEM
E_RAW = 320000
E_SL = E_RAW + N_NODES
P = 8192

NC = 2
NS = 16
NW = NC * NS
L = 16

E_PAD = 344064
EPW = E_PAD // NW
G = 64
F_PT = 4
ECH = 4096
NCHB = E_PAD // ECH
NDV = N_NODES // L
ND_PAD = 10240
NDR = ND_PAD // 128
ROWS_PT = N_NODES // NW
ROWS_PS = ND_PAD // NS
ZB = 128
B_SCORE = 4 * P
BPW = B_SCORE // NW
GC = 128
NCHC = BPW // GC

_mesh = plsc.VectorSubcoreMesh(core_axis_name="c", subcore_axis_name="s")


def _fori(lo, hi, body):
    lax.fori_loop(jnp.int32(lo), jnp.int32(hi), body, 0)


def _tc_pre_body(fu_ref, fi_ref, wu_ref, bu_ref, wi_ref, bi_ref, wg_ref,
                 aa_ref, h0_ref, elr_ref, c_ref):
    wg = wg_ref[...]
    m_u = jnp.dot(wu_ref[...], wg, preferred_element_type=jnp.float32)
    m_i = jnp.dot(wi_ref[...], wg, preferred_element_type=jnp.float32)
    cb_u = jnp.dot(bu_ref[...], wg, preferred_element_type=jnp.float32)
    cb_i = jnp.dot(bi_ref[...], wg, preferred_element_type=jnp.float32)
    h0u = jnp.dot(fu_ref[...], m_u, preferred_element_type=jnp.float32) + cb_u
    h0i = jnp.dot(fi_ref[...], m_i, preferred_element_type=jnp.float32) + cb_i
    h0_ref[0:N_USER, :] = h0u
    h0_ref[N_USER:N_NODES, :] = h0i
    aa = aa_ref[...]
    elr_u = lax.dot_general(aa, h0u, (((1,), (1,)), ((), ())),
                            preferred_element_type=jnp.float32)
    elr_i = lax.dot_general(aa, h0i, (((1,), (1,)), ((), ())),
                            preferred_element_type=jnp.float32)
    elr_ref[:, 0:N_USER] = elr_u
    elr_ref[:, N_USER:N_NODES] = elr_i
    mx = (jnp.maximum(jnp.max(elr_u[0]), jnp.max(elr_i[0]))
          + jnp.maximum(jnp.max(elr_u[1]), jnp.max(elr_i[1])))
    c_ref[...] = jnp.where(mx > 0, mx, 0.2 * mx).reshape(1, 1)


def _tc_pre(feat_user, feat_item, w_u, b_u, w_i, b_i, w_g, attn2):
    return pl.pallas_call(
        _tc_pre_body,
        out_shape=[
            jax.ShapeDtypeStruct((N_NODES, 128), jnp.float32),
            jax.ShapeDtypeStruct((2, N_NODES), jnp.float32),
            jax.ShapeDtypeStruct((1, 1), jnp.float32),
        ],
    )(feat_user, feat_item, w_u, b_u, w_i, b_i, w_g, attn2)


@functools.partial(
    pl.kernel,
    out_type=[jax.ShapeDtypeStruct((NC, NDR, 128), jnp.float32),
              jax.ShapeDtypeStruct((E_PAD,), jnp.float32)],
    mesh=_mesh,
    compiler_params=pltpu.CompilerParams(needs_layout_passes=False),
    scratch_types=[
        pltpu.VMEM((N_NODES,), jnp.float32),
        pltpu.VMEM((N_NODES,), jnp.float32),
        pltpu.VMEM((NDR, 128), jnp.float32),
        pltpu.VMEM((EPW,), jnp.int32),
        pltpu.VMEM((EPW,), jnp.int32),
        pltpu.VMEM((EPW,), jnp.float32),
        pltpu.VMEM((L,), jnp.float32),
        pltpu.VMEM((NDR,), jnp.int32),
        pltpu.VMEM_SHARED((NDR, 128), jnp.float32),
    ],
)
def _sc_a(src_hbm, dst_hbm, el_hbm, er_hbm, c_hbm, den_out, ee_out,
          el_v, er_v, den_v, src_v, dst_v, ee_v, c_v, iota_v, shared_den):
    c = lax.axis_index("c").astype(jnp.int32)
    s = lax.axis_index("s").astype(jnp.int32)
    wid = s * jnp.int32(NC) + c
    base = wid * jnp.int32(EPW)
    pltpu.sync_copy(el_hbm, el_v)
    pltpu.sync_copy(er_hbm, er_v)
    pltpu.sync_copy(c_hbm, c_v)
    pltpu.sync_copy(src_hbm.at[pl.ds(base, EPW)], src_v)
    pltpu.sync_copy(dst_hbm.at[pl.ds(base, EPW)], dst_v)

    zero = jnp.zeros((L,), jnp.float32)
    iota = lax.iota(jnp.int32, L)

    def zbody(i, carry):
        for k in range(8):
            den_v[i, pl.ds(k * L, L)] = zero
        return carry
    _fori(0, NDR, zbody)

    def ibody(i, carry):
        iota_v[pl.ds(i * jnp.int32(L), L)] = iota + i * jnp.int32(L)
        return carry
    _fori(0, NDR // L, ibody)

    @pl.when(s == 0)
    def _():
        pltpu.sync_copy(den_v, shared_den)
    plsc.subcore_barrier()

    cvec = c_v[...]

    def ebody(i, carry):
        off = i * jnp.int32(L)
        sv = src_v[pl.ds(off, L)]
        dv = dst_v[pl.ds(off, L)]
        e = plsc.load_gather(el_v, [sv]) + plsc.load_gather(er_v, [dv])
        e = jnp.where(e > 0, e, 0.2 * e)
        ee = jnp.exp(e - cvec)
        gidx = (base + off) + iota
        ee = jnp.where(gidx < jnp.int32(E_SL), ee, 0.0)
        ee_v[pl.ds(off, L)] = ee
        plsc.addupdate_scatter(
            den_v, [lax.shift_right_logical(dv, jnp.int32(7)),
                    lax.bitwise_and(dv, jnp.int32(127))], ee)
        return carry
    _fori(0, EPW // L, ebody)

    pltpu.sync_copy(ee_v, ee_out.at[pl.ds(base, EPW)])
    pltpu.sync_copy(den_v, shared_den.at[iota_v], add=True)
    plsc.subcore_barrier()

    @pl.when(s == 0)
    def _():
        pltpu.sync_copy(shared_den, den_out.at[c])


@functools.partial(
    pl.kernel,
    out_type=jax.ShapeDtypeStruct((NW, F_PT, ND_PAD), jnp.float32),
    mesh=_mesh,
    compiler_params=pltpu.CompilerParams(needs_layout_passes=False),
    scratch_types=[
        pltpu.VMEM((F_PT, ND_PAD), jnp.float32),
        pltpu.VMEM((F_PT, ND_PAD), jnp.float32),
        pltpu.VMEM((ECH,), jnp.int32),
        pltpu.VMEM((ECH,), jnp.float32),
        pltpu.VMEM((ECH,), jnp.int32),
        pltpu.VMEM((ECH,), jnp.float32),
        pltpu.SemaphoreType.DMA,
        pltpu.SemaphoreType.DMA,
    ],
)
def _sc_b(pk_hbm, ee_hbm, h0t_hbm,
          agg_out, h0t_v, acc_v, pk_a, ee_a, pk_b, ee_b, sa, sb):
    c = lax.axis_index("c").astype(jnp.int32)
    s = lax.axis_index("s").astype(jnp.int32)
    wid = s * jnp.int32(NC) + c
    pltpu.sync_copy(h0t_hbm.at[wid], h0t_v)

    zero = jnp.zeros((L,), jnp.float32)

    def zbody(i, carry):
        for f in range(F_PT):
            acc_v[f, pl.ds(i * jnp.int32(L), L)] = zero
        return carry
    _fori(0, ND_PAD // L, zbody)

    fvec = [jnp.full((L,), f, jnp.int32) for f in range(F_PT)]

    def process(pk_v, ee_v):
        UNR = 8

        def ib(i, carry):
            off = i * jnp.int32(UNR * L)
            svs, dvs, ees = [], [], []
            for u in range(UNR):
                o = off + u * L
                pk = pk_v[pl.ds(o, L)]
                ees.append(ee_v[pl.ds(o, L)])
                svs.append(lax.shift_right_logical(pk, jnp.int32(14)))
                dvs.append(lax.bitwise_and(pk, jnp.int32(16383)))
            hvs = [plsc.load_gather(h0t_v, [fvec[f], svs[u]])
                   for u in range(UNR) for f in range(F_PT)]
            ms = [hvs[u * F_PT + f] * ees[u]
                  for u in range(UNR) for f in range(F_PT)]
            for u in range(UNR):
                for f in range(F_PT):
                    plsc.addupdate_scatter(
                        acc_v, [fvec[f], dvs[u]], ms[u * F_PT + f])
            return carry
        _fori(0, ECH // (UNR * L), ib)

    def stage(ch, pk_v, ee_v, sem):
        e0 = ch * jnp.int32(ECH)
        pltpu.async_copy(pk_hbm.at[pl.ds(e0, ECH)], pk_v, sem)
        pltpu.async_copy(ee_hbm.at[pl.ds(e0, ECH)], ee_v, sem)

    def swait(pk_v, ee_v, sem):
        pltpu.make_async_copy(pk_hbm.at[pl.ds(0, ECH)], pk_v, sem).wait()
        pltpu.make_async_copy(ee_hbm.at[pl.ds(0, ECH)], ee_v, sem).wait()

    stage(jnp.int32(0), pk_a, ee_a, sa)
    NP = NCHB // 2

    def pbody(i, carry):
        cha = i * jnp.int32(2)
        chb = cha + jnp.int32(1)
        swait(pk_a, ee_a, sa)
        stage(chb, pk_b, ee_b, sb)
        process(pk_a, ee_a)
        swait(pk_b, ee_b, sb)

        @pl.when(i < NP - 1)
        def _():
            stage(cha + jnp.int32(2), pk_a, ee_a, sa)
        process(pk_b, ee_b)
        return carry
    _fori(0, NP, pbody)

    pltpu.sync_copy(acc_v, agg_out.at[wid])


def _tc_mid_body(p_ref, den_ref, bg_ref, wp_ref, bp_ref, h_ref, out_ref):
    sfull = p_ref[...] / den_ref[...] + bg_ref[...]
    h = jnp.where(sfull > 0, sfull, jnp.exp(jnp.minimum(sfull, 0.0)) - 1.0)
    h_ref[...] = h
    out_ref[...] = (jnp.dot(h[0:N_USER, :], wp_ref[...],
                            preferred_element_type=jnp.float32) + bp_ref[...])


def _tc_mid(agg2, den, b_g, w_p, b_p):
    return pl.pallas_call(
        _tc_mid_body,
        out_shape=[
            jax.ShapeDtypeStruct((N_NODES, 128), jnp.float32),
            jax.ShapeDtypeStruct((N_USER, 64), jnp.float32),
        ],
    )(agg2, den, b_g, w_p, b_p)


@functools.partial(
    pl.kernel,
    out_type=jax.ShapeDtypeStruct((B_SCORE, 128), jnp.float32),
    mesh=_mesh,
    compiler_params=pltpu.CompilerParams(needs_layout_passes=False),
    scratch_types=[
        pltpu.VMEM((NCHC, GC), jnp.int32),
        pltpu.VMEM((GC, 128), jnp.float32),
    ],
)
def _sc_c(idx3_hbm, h_hbm, rows_out, idx_v, rows_v):
    c = lax.axis_index("c").astype(jnp.int32)
    s = lax.axis_index("s").astype(jnp.int32)
    wid = s * jnp.int32(NC) + c
    pltpu.sync_copy(idx3_hbm.at[wid], idx_v)

    def chunk(ch, carry):
        pltpu.sync_copy(h_hbm.at[idx_v.at[ch]], rows_v)
        pltpu.sync_copy(rows_v,
                        rows_out.at[pl.ds(wid * jnp.int32(BPW) + ch * jnp.int32(GC), GC)])
        return carry
    _fori(0, NCHC, chunk)


def _tc_loss_body(rows_ref, at_ref, loss_ref):
    ps = jnp.sum(rows_ref[0] * rows_ref[1], axis=1)
    ns = jnp.sum(rows_ref[2] * rows_ref[3], axis=1)
    k_t = jnp.minimum(jnp.float32(0.8), at_ref[0, 0])

    def part(sc):
        l = jnp.minimum(sc, 0.0) - jnp.log(1.0 + jnp.exp(-jnp.abs(sc)))
        hold = k_t * jnp.max(l)
        l = jnp.where(l > hold, 0.0, l)
        return -jnp.sum(l)

    loss_ref[...] = (part(ps) + part(ns)).reshape(1, 1)


def _tc_loss(rows4, alpha_t):
    return pl.pallas_call(
        _tc_loss_body,
        out_shape=jax.ShapeDtypeStruct((1, 1), jnp.float32),
    )(rows4, alpha_t)


def kernel(feat_user, feat_item, edge_index, pos_src, pos_dst, neg_src,
           neg_dst, alpha_T, W_user, b_user, W_item, b_item, W_gat,
           attn_l, attn_r, b_gat, W_pred, b_pred):
    f32 = jnp.float32
    i32 = jnp.int32
    loop = jnp.arange(N_NODES, dtype=i32)
    pad = jnp.zeros((E_PAD - E_SL,), dtype=i32)
    src = jnp.concatenate([edge_index[0].astype(i32), loop, pad])
    dst = jnp.concatenate([edge_index[1].astype(i32), loop, pad])
    attn2 = jnp.stack([attn_l, attn_r], axis=0).astype(f32)

    h0, elr, c11 = _tc_pre(
        feat_user.astype(f32), feat_item.astype(f32),
        W_user.astype(f32), b_user.reshape(1, -1).astype(f32),
        W_item.astype(f32), b_item.reshape(1, -1).astype(f32),
        W_gat.astype(f32), attn2)
    el = elr[0]
    er = elr[1]
    c16 = jnp.broadcast_to(c11.reshape(()), (L,))

    den2, ee = _sc_a(src, dst, el, er, c16)

    packed = src * jnp.int32(16384) + dst
    h0p = jnp.zeros((ND_PAD, 128), jnp.float32).at[:N_NODES].set(h0)
    h0t = h0p.T.reshape(NW, F_PT, ND_PAD)
    agg4 = _sc_b(packed, ee, h0t)

    psum = agg4.reshape(128, ND_PAD)[:, :N_NODES].T
    den = (den2[0] + den2[1]).reshape(ND_PAD)[:N_NODES].reshape(N_NODES, 1)

    h, out_pred = _tc_mid(psum, den, b_gat.reshape(1, -1).astype(f32),
                          W_pred.astype(f32), b_pred.reshape(1, -1).astype(f32))

    idx3 = jnp.concatenate([pos_src, pos_dst, neg_src, neg_dst]
                           ).astype(i32).reshape(NW, NCHC, GC)
    rows = _sc_c(idx3, h)
    rows4 = rows.reshape(4, P, 128)

    loss11 = _tc_loss(rows4, alpha_T.reshape(1, 1).astype(f32))
    return (loss11[0, 0], out_pred)
